# baseline (speedup 1.0000x reference)
import jax
import jax.numpy as jnp
from jax.experimental import pallas as pl

K = 20


def _lin(x, W, b):
    return x @ W + b


def _bn(x, g, bt):
    axes = tuple(range(x.ndim - 1))
    m = jnp.mean(x, axis=axes, keepdims=True)
    v = jnp.var(x, axis=axes, keepdims=True)
    return (x - m) / jnp.sqrt(v + 1e-5) * g + bt


def _mlp(x, layers):
    for (W, b, g, bt) in layers:
        x = _bn(jax.nn.relu(_lin(x, W, b)), g, bt)
    return x


def _pos(x, layers):
    for (W, b, g, bt) in layers:
        x = jax.nn.relu(_bn(_lin(x, W, b), g, bt))
    return x


def _dyn_edge_conv(x, layers):
    sq = jnp.sum(x * x, axis=-1)
    d2 = sq[:, :, None] + sq[:, None, :] - 2.0 * jnp.einsum('bnd,bmd->bnm', x, x)
    _, idx = jax.lax.top_k(-d2, K)
    xj = jax.vmap(lambda xb, ib: xb[ib])(x, idx)
    xi = jnp.broadcast_to(x[:, :, None, :], xj.shape)
    e = jnp.concatenate([xi, xj - xi], axis=-1)
    return jnp.max(_mlp(e, layers), axis=2)


def _se(x, p):
    W1, b1, W2, b2 = p
    avg = jnp.mean(x, axis=1)
    attn = jax.nn.sigmoid(_lin(jax.nn.relu(_lin(avg, W1, b1)), W2, b2))
    return x * attn[:, None, :]


def _branch(x, params):
    pos1 = _pos(x[..., :3], params['pos1'])
    h = _dyn_edge_conv(x, params['conv1']) + pos1
    h = _se(h, params['se1'])
    pos2 = _pos(x[..., :3], params['pos2'])
    h2 = _dyn_edge_conv(h, params['conv2']) + pos2
    h2 = _se(h2, params['se2'])
    return h2


def _copy_kernel(x_ref, o_ref):
    o_ref[...] = x_ref[...]


def kernel(x, x2, batch, batch2, y, params):
    Bn = y.shape[0]
    xa = x.reshape(Bn, -1, 6)
    xb = x2.reshape(Bn, -1, 6)
    h1 = _branch(xa, params)
    h2 = _branch(xb, params)
    g1 = jnp.max(h1, axis=1)
    g2 = jnp.max(h2, axis=1)
    d = g2 - g1
    h = _mlp(d, [params['mlp2_l1']])
    out = _lin(h, *params['mlp2_out'])
    out = pl.pallas_call(
        _copy_kernel,
        out_shape=jax.ShapeDtypeStruct(out.shape, out.dtype),
    )(out)
    return jax.nn.log_softmax(out, axis=-1)


# speedup vs baseline: 4.0006x; 4.0006x over previous
import functools

import jax
import jax.numpy as jnp
from jax import lax
from jax.experimental import pallas as pl
from jax.experimental.pallas import tpu as pltpu
from jax.experimental.pallas import tpu_sc as plsc

G = 8
N = 1024
NK = 20
T = G * N
E = NK * T
EPS = 1e-5
F32 = jnp.float32
I32 = jnp.int32
INT_MAX = 0x7FFFFFFF
HI = lax.Precision.HIGHEST


def _sortable_key(d2):
    b = lax.bitcast_convert_type(d2, I32)
    return b ^ lax.shift_right_arithmetic(b, 31) & INT_MAX


def _bn_apply(x, m, s, gg, tt):
    return (x - m) / s * gg + tt


def _k1_body(xr_ref, xT_ref, w1_ref, b1_ref, r1_ref, st_ref):
    g = pl.program_id(0)
    t = pl.program_id(1)

    xr = xr_ref[0]
    xT = xT_ref[0]
    sqr = jnp.sum(xr * xr, axis=1, keepdims=True)
    sqg = jnp.sum(xT * xT, axis=0, keepdims=True)
    d2 = sqr + sqg - 2.0 * jnp.dot(xr, xT, preferred_element_type=F32)
    R = d2.shape[0]
    key = _sortable_key(d2)
    iota = lax.broadcasted_iota(I32, (R, N), 1)

    xg = xT.T
    xi = xr[:, :6]

    s_sum = jnp.zeros((1, 64), F32)
    s_sq = jnp.zeros((1, 64), F32)
    for k in range(NK):
        m = jnp.min(key, axis=1, keepdims=True)
        idxv = jnp.min(jnp.where(key == m, iota, N), axis=1, keepdims=True)
        oh = (iota == idxv)
        ohf = jnp.where(oh, 1.0, 0.0).astype(F32)
        xj = jnp.dot(ohf, xg, preferred_element_type=F32, precision=HI)
        e = jnp.concatenate([xi, xj[:, :6] - xi], axis=1)
        u = jnp.maximum(
            jnp.dot(e, w1_ref[...], preferred_element_type=F32) + b1_ref[...], 0.0)
        r1_ref[k] = u
        s_sum = s_sum + jnp.sum(u, axis=0, keepdims=True)
        s_sq = s_sq + jnp.sum(u * u, axis=0, keepdims=True)
        key = jnp.where(oh, INT_MAX, key)

    @pl.when((g == 0) & (t == 0))
    def _():
        st_ref[...] = jnp.zeros_like(st_ref)

    st_ref[...] += jnp.concatenate([s_sum, s_sq], axis=0)


def _k1_call(x3d, xT3d, w1, b1, R):
    return pl.pallas_call(
        _k1_body,
        grid=(G, N // R),
        in_specs=[
            pl.BlockSpec((1, R, 16), lambda g, t: (g, t, 0)),
            pl.BlockSpec((1, 16, N), lambda g, t: (g, 0, 0)),
            pl.BlockSpec((12, 64), lambda g, t: (0, 0)),
            pl.BlockSpec((1, 64), lambda g, t: (0, 0)),
        ],
        out_specs=[
            pl.BlockSpec((NK, R, 64), lambda g, t: (0, g * (N // R) + t, 0)),
            pl.BlockSpec((2, 64), lambda g, t: (0, 0)),
        ],
        out_shape=[
            jax.ShapeDtypeStruct((NK, T, 64), F32),
            jax.ShapeDtypeStruct((2, 64), F32),
        ],
    )(x3d, xT3d, w1, b1)


def _k4_body(r1_ref, m_ref, s_ref, g_ref, t_ref, w_ref, b_ref, r2_ref, st_ref):
    y = _bn_apply(r1_ref[...], m_ref[...], s_ref[...], g_ref[...], t_ref[...])
    u = jnp.maximum(jnp.dot(y, w_ref[...], preferred_element_type=F32)
                    + b_ref[...], 0.0)
    r2_ref[...] = u

    @pl.when(pl.program_id(0) == 0)
    def _():
        st_ref[...] = jnp.zeros_like(st_ref)

    st_ref[...] += jnp.concatenate(
        [jnp.sum(u, axis=0, keepdims=True), jnp.sum(u * u, axis=0, keepdims=True)],
        axis=0)


def _k4_call(r1_flat, m, s, gg, tt, w, b, Re):
    small = [pl.BlockSpec((1, 64), lambda i: (0, 0))] * 4
    return pl.pallas_call(
        _k4_body,
        grid=(E // Re,),
        in_specs=[pl.BlockSpec((Re, 64), lambda i: (i, 0))] + small + [
            pl.BlockSpec((64, 64), lambda i: (0, 0)),
            pl.BlockSpec((1, 64), lambda i: (0, 0)),
        ],
        out_specs=[
            pl.BlockSpec((Re, 64), lambda i: (i, 0)),
            pl.BlockSpec((2, 64), lambda i: (0, 0)),
        ],
        out_shape=[
            jax.ShapeDtypeStruct((E, 64), F32),
            jax.ShapeDtypeStruct((2, 64), F32),
        ],
    )(r1_flat, m, s, gg, tt, w, b)


def _k5_body(r2_ref, m_ref, s_ref, g_ref, t_ref, w_ref, b_ref,
             mx_ref, mn_ref, st_ref):
    s_sum = jnp.zeros((1, 64), F32)
    s_sq = jnp.zeros((1, 64), F32)
    mx = None
    for k in range(NK):
        y = _bn_apply(r2_ref[k], m_ref[...], s_ref[...], g_ref[...], t_ref[...])
        u = jnp.maximum(jnp.dot(y, w_ref[...], preferred_element_type=F32)
                        + b_ref[...], 0.0)
        s_sum = s_sum + jnp.sum(u, axis=0, keepdims=True)
        s_sq = s_sq + jnp.sum(u * u, axis=0, keepdims=True)
        if mx is None:
            mx, mn = u, u
        else:
            mx = jnp.maximum(mx, u)
            mn = jnp.minimum(mn, u)
    mx_ref[...] = mx
    mn_ref[...] = mn

    @pl.when(pl.program_id(0) == 0)
    def _():
        st_ref[...] = jnp.zeros_like(st_ref)

    st_ref[...] += jnp.concatenate([s_sum, s_sq], axis=0)


def _k5_call(r2, m, s, gg, tt, w, b, R):
    small = [pl.BlockSpec((1, 64), lambda i: (0, 0))] * 4
    return pl.pallas_call(
        _k5_body,
        grid=(T // R,),
        in_specs=[pl.BlockSpec((NK, R, 64), lambda i: (0, i, 0))] + small + [
            pl.BlockSpec((64, 64), lambda i: (0, 0)),
            pl.BlockSpec((1, 64), lambda i: (0, 0)),
        ],
        out_specs=[
            pl.BlockSpec((R, 64), lambda i: (i, 0)),
            pl.BlockSpec((R, 64), lambda i: (i, 0)),
            pl.BlockSpec((2, 64), lambda i: (0, 0)),
        ],
        out_shape=[
            jax.ShapeDtypeStruct((T, 64), F32),
            jax.ShapeDtypeStruct((T, 64), F32),
            jax.ShapeDtypeStruct((2, 64), F32),
        ],
    )(r2, m, s, gg, tt, w, b)


def _kp_body(x_ref, w11, b11, g11, t11, w12, b12, g12, t12,
             w21, b21, g21, t21, w22, b22, g22, t22, p1_ref, p2_ref):
    x3 = x_ref[:, :3]

    def block(h, w, b, gg, tt):
        u = jnp.dot(h, w[...], preferred_element_type=F32) + b[...]
        m = jnp.mean(u, axis=0, keepdims=True)
        c = u - m
        v = jnp.mean(c * c, axis=0, keepdims=True)
        return jnp.maximum(c / jnp.sqrt(v + EPS) * gg[...] + tt[...], 0.0)

    h = block(x3, w11, b11, g11, t11)
    p1_ref[...] = block(h, w12, b12, g12, t12)
    h = block(x3, w21, b21, g21, t21)
    p2_ref[...] = block(h, w22, b22, g22, t22)


def _kp_call(x16, pos1, pos2):
    args = [x16]
    for (w, b, gg, tt) in list(pos1) + list(pos2):
        args += [w, b.reshape(1, -1), gg.reshape(1, -1), tt.reshape(1, -1)]
    return pl.pallas_call(
        _kp_body,
        out_shape=[
            jax.ShapeDtypeStruct((T, 64), F32),
            jax.ShapeDtypeStruct((T, 256), F32),
        ],
    )(*args)


def _k6_body(mx_ref, mn_ref, pos1_ref, m_ref, s_ref, g_ref, t_ref,
             sw1, sb1, sw2, sb2, h_ref):
    sel = g_ref[...] >= 0.0
    r = jnp.where(sel, mx_ref[...], mn_ref[...])
    conv = _bn_apply(r, m_ref[...], s_ref[...], g_ref[...], t_ref[...])
    h = conv + pos1_ref[...]
    h3 = h.reshape(G, N, 64)
    avg = jnp.mean(h3, axis=1)
    a = jnp.maximum(jnp.dot(avg, sw1[...], preferred_element_type=F32) + sb1[...], 0.0)
    attn = jax.nn.sigmoid(jnp.dot(a, sw2[...], preferred_element_type=F32) + sb2[...])
    h_ref[...] = (h3 * attn[:, None, :]).reshape(T, 64)


def _k6_call(mx, mn, pos1, m, s, gg, tt, se1):
    sw1, sb1, sw2, sb2 = se1
    return pl.pallas_call(
        _k6_body,
        out_shape=jax.ShapeDtypeStruct((T, 64), F32),
    )(mx, mn, pos1, m, s, gg, tt, sw1, sb1.reshape(1, -1), sw2, sb2.reshape(1, -1))


def _k7_body(hg_ref, hTr_ref, idx_ref):
    g = pl.program_id(0)
    hg = hg_ref[0]
    hTr = hTr_ref[0]
    sqg = jnp.sum(hg * hg, axis=1, keepdims=True)
    sqr = jnp.sum(hTr * hTr, axis=0, keepdims=True)
    d2 = sqg + sqr - 2.0 * jnp.dot(hg, hTr, preferred_element_type=F32)
    R = d2.shape[1]
    key = _sortable_key(d2)
    iota = lax.broadcasted_iota(I32, (N, R), 0)
    base = g * N
    for k in range(NK):
        m = jnp.min(key, axis=0, keepdims=True)
        idxv = jnp.min(jnp.where(key == m, iota, N), axis=0, keepdims=True)
        idx_ref[k] = idxv[0] + base
        key = jnp.where(iota == idxv, INT_MAX, key)


def _k7_call(h3d, hT3d, R):
    return pl.pallas_call(
        _k7_body,
        grid=(G, N // R),
        in_specs=[
            pl.BlockSpec((1, N, 64), lambda g, t: (g, 0, 0)),
            pl.BlockSpec((1, 64, R), lambda g, t: (g, 0, t)),
        ],
        out_specs=pl.BlockSpec((NK, R), lambda g, t: (0, g * (N // R) + t)),
        out_shape=jax.ShapeDtypeStruct((NK, T), I32),
    )(h3d, hT3d)


_SC_CH = 256


def _sc_gather(table, idx_flat):
    info = plsc.get_sparse_core_info()
    nw = info.num_cores * info.num_subcores
    bpw = E // nw
    mesh = plsc.VectorSubcoreMesh(core_axis_name="c", subcore_axis_name="s")

    @functools.partial(
        pl.kernel,
        out_type=jax.ShapeDtypeStruct((E, 128), F32),
        mesh=mesh,
        scratch_types=[
            pltpu.VMEM((bpw,), I32),
            pltpu.VMEM((_SC_CH, 128), F32),
            pltpu.SemaphoreType.DMA,
        ],
    )
    def k8(tab_hbm, idx_hbm, out_hbm, idx_v, rows_v, sem):
        wid = lax.axis_index("s") * info.num_cores + lax.axis_index("c")
        base = wid * bpw
        pltpu.sync_copy(idx_hbm.at[pl.ds(base, bpw)], idx_v)

        def chunk(c, carry):
            pltpu.async_copy(
                tab_hbm.at[idx_v.at[pl.ds(c * _SC_CH, _SC_CH)]], rows_v, sem
            ).wait()
            pltpu.sync_copy(rows_v, out_hbm.at[pl.ds(base + c * _SC_CH, _SC_CH)])
            return carry

        lax.fori_loop(0, bpw // _SC_CH, chunk, 0)

    return k8(table, idx_flat)


def _k9_body(hj_ref, h_ref, w_ref, b_ref, mx_ref, mn_ref, st_ref):
    hi = h_ref[:, :64]
    s_sum = jnp.zeros((1, 256), F32)
    s_sq = jnp.zeros((1, 256), F32)
    mx = None
    for k in range(NK):
        e = jnp.concatenate([hi, hj_ref[k][:, :64] - hi], axis=1)
        t = jnp.maximum(jnp.dot(e, w_ref[...], preferred_element_type=F32)
                        + b_ref[...], 0.0)
        s_sum = s_sum + jnp.sum(t, axis=0, keepdims=True)
        s_sq = s_sq + jnp.sum(t * t, axis=0, keepdims=True)
        if mx is None:
            mx, mn = t, t
        else:
            mx = jnp.maximum(mx, t)
            mn = jnp.minimum(mn, t)
    mx_ref[...] = mx
    mn_ref[...] = mn

    @pl.when(pl.program_id(0) == 0)
    def _():
        st_ref[...] = jnp.zeros_like(st_ref)

    st_ref[...] += jnp.concatenate([s_sum, s_sq], axis=0)


def _k9_call(hj, h128, w, b, R):
    return pl.pallas_call(
        _k9_body,
        grid=(T // R,),
        in_specs=[
            pl.BlockSpec((NK, R, 128), lambda i: (0, i, 0)),
            pl.BlockSpec((R, 128), lambda i: (i, 0)),
            pl.BlockSpec((128, 256), lambda i: (0, 0)),
            pl.BlockSpec((1, 256), lambda i: (0, 0)),
        ],
        out_specs=[
            pl.BlockSpec((R, 256), lambda i: (i, 0)),
            pl.BlockSpec((R, 256), lambda i: (i, 0)),
            pl.BlockSpec((2, 256), lambda i: (0, 0)),
        ],
        out_shape=[
            jax.ShapeDtypeStruct((T, 256), F32),
            jax.ShapeDtypeStruct((T, 256), F32),
            jax.ShapeDtypeStruct((2, 256), F32),
        ],
    )(hj, h128, w, b)


def _k10_body(mx_ref, mn_ref, pos2_ref, m_ref, s_ref, g_ref, t_ref,
              sw1, sb1, sw2, sb2, g_out_ref):
    sel = g_ref[...] >= 0.0
    r = jnp.where(sel, mx_ref[...], mn_ref[...])
    h = _bn_apply(r, m_ref[...], s_ref[...], g_ref[...], t_ref[...]) + pos2_ref[...]
    h3 = h.reshape(G, N, 256)
    avg = jnp.mean(h3, axis=1)
    a = jnp.maximum(jnp.dot(avg, sw1[...], preferred_element_type=F32) + sb1[...], 0.0)
    attn = jax.nn.sigmoid(jnp.dot(a, sw2[...], preferred_element_type=F32) + sb2[...])
    g_out_ref[...] = jnp.max(h3 * attn[:, None, :], axis=1)


def _k10_call(mx, mn, pos2, m, s, gg, tt, se2):
    sw1, sb1, sw2, sb2 = se2
    return pl.pallas_call(
        _k10_body,
        out_shape=jax.ShapeDtypeStruct((G, 256), F32),
    )(mx, mn, pos2, m, s, gg, tt, sw1, sb1.reshape(1, -1), sw2, sb2.reshape(1, -1))


def _k11_body(g1_ref, g2_ref, w_ref, b_ref, gg_ref, tt_ref, wo_ref, bo_ref, o_ref):
    d = g2_ref[...] - g1_ref[...]
    u = jnp.maximum(jnp.dot(d, w_ref[...], preferred_element_type=F32) + b_ref[...], 0.0)
    m = jnp.mean(u, axis=0, keepdims=True)
    c = u - m
    v = jnp.mean(c * c, axis=0, keepdims=True)
    hb = c / jnp.sqrt(v + EPS) * gg_ref[...] + tt_ref[...]
    o = jnp.dot(hb, wo_ref[...], preferred_element_type=F32) + bo_ref[...]
    mo = jnp.max(o, axis=1, keepdims=True)
    z = o - mo
    o_ref[...] = z - jnp.log(jnp.sum(jnp.exp(z), axis=1, keepdims=True))


def _k11_call(g1, g2, l1, lout):
    w, b, gg, tt = l1
    wo, bo = lout
    return pl.pallas_call(
        _k11_body,
        out_shape=jax.ShapeDtypeStruct((G, 5), F32),
    )(g1, g2, w, b.reshape(1, -1), gg.reshape(1, -1), tt.reshape(1, -1),
      wo, bo.reshape(1, -1))


def _bn_ms(st, count):
    m = st[0:1] / count
    v = st[1:2] / count - m * m
    return m, jnp.sqrt(v + EPS)


def _branch_fast(xf, p):
    x16 = jnp.pad(xf, ((0, 0), (0, 10)))
    x3d = x16.reshape(G, N, 16)
    xT3d = jnp.transpose(x3d, (0, 2, 1))

    pos1, pos2 = _kp_call(x16, p['pos1'], p['pos2'])

    (w1, b1, g1, t1), (w2, b2, g2, t2), (w3, b3, g3, t3) = p['conv1']
    r1, st1 = _k1_call(x3d, xT3d, w1, b1.reshape(1, -1), 256)

    m1, s1 = _bn_ms(st1, float(E))
    r2_flat, st2 = _k4_call(r1.reshape(E, 64), m1, s1, g1.reshape(1, -1),
                            t1.reshape(1, -1), w2, b2.reshape(1, -1), 2048)

    m2, s2 = _bn_ms(st2, float(E))
    mx3, mn3, st3 = _k5_call(r2_flat.reshape(NK, T, 64), m2, s2, g2.reshape(1, -1),
                             t2.reshape(1, -1), w3, b3.reshape(1, -1), 512)

    m3, s3 = _bn_ms(st3, float(E))
    h = _k6_call(mx3, mn3, pos1, m3, s3, g3.reshape(1, -1), t3.reshape(1, -1),
                 p['se1'])

    h3d = h.reshape(G, N, 64)
    hT3d = jnp.transpose(h3d, (0, 2, 1))
    idx2 = _k7_call(h3d, hT3d, 256)

    h128 = jnp.pad(h, ((0, 0), (0, 64)))
    hj = _sc_gather(h128, idx2.reshape(E))

    (w2c, b2c, g2c, t2c), = p['conv2']
    mx2, mn2, st4 = _k9_call(hj.reshape(NK, T, 128), h128, w2c,
                             b2c.reshape(1, -1), 256)

    m4, s4 = _bn_ms(st4, float(E))
    return _k10_call(mx2, mn2, pos2, m4, s4, g2c.reshape(1, -1),
                     t2c.reshape(1, -1), p['se2'])


def kernel(x, x2, batch, batch2, y, params):
    del batch, batch2, y
    gA = _branch_fast(x, params)
    gB = _branch_fast(x2, params)
    return _k11_call(gA, gB, params['mlp2_l1'], params['mlp2_out'])


# speedup vs baseline: 8.7721x; 2.1927x over previous
import functools

import jax
import jax.numpy as jnp
from jax import lax
from jax.experimental import pallas as pl
from jax.experimental.pallas import tpu as pltpu
from jax.experimental.pallas import tpu_sc as plsc

G = 8
N = 1024
NK = 20
T = G * N
E = NK * T
EPS = 1e-5
F32 = jnp.float32
I32 = jnp.int32
INT_MAX = 0x7FFFFFFF
HI = lax.Precision.HIGHEST


def _sortable_key(d2):
    b = lax.bitcast_convert_type(d2, I32)
    return b ^ lax.shift_right_arithmetic(b, 31) & INT_MAX


def _bn_apply(x, m, s, gg, tt):
    return (x - m) / s * gg + tt


def _ksel_body(hg_ref, hTr_ref, idx_ref):
    g = pl.program_id(0)
    hg = hg_ref[0]
    hTr = hTr_ref[0]
    sqg = jnp.sum(hg * hg, axis=1, keepdims=True)
    sqr = jnp.sum(hTr * hTr, axis=0, keepdims=True)
    d2 = sqg + sqr - 2.0 * jnp.dot(hg, hTr, preferred_element_type=F32)
    R = d2.shape[1]
    key = _sortable_key(d2)
    iota = lax.broadcasted_iota(I32, (N, R), 0)
    base = g * N
    for k in range(NK):
        m = jnp.min(key, axis=0, keepdims=True)
        idxv = jnp.min(jnp.where(key == m, iota, N), axis=0, keepdims=True)
        idx_ref[k] = idxv[0] + base
        key = jnp.where(iota == idxv, INT_MAX, key)


def _ksel_call(h3d, hT3d, R):
    D = h3d.shape[-1]
    return pl.pallas_call(
        _ksel_body,
        grid=(G, N // R),
        in_specs=[
            pl.BlockSpec((1, N, D), lambda g, t: (g, 0, 0)),
            pl.BlockSpec((1, D, R), lambda g, t: (g, 0, t)),
        ],
        out_specs=pl.BlockSpec((NK, R), lambda g, t: (0, g * (N // R) + t)),
        out_shape=jax.ShapeDtypeStruct((NK, T), I32),
    )(h3d, hT3d)


def _k3_body(xj_ref, xr_ref, w1_ref, b1_ref, r1_ref, st_ref):
    xi = xr_ref[:, :6]
    s_sum = jnp.zeros((1, 64), F32)
    s_sq = jnp.zeros((1, 64), F32)
    for k in range(NK):
        e = jnp.concatenate([xi, xj_ref[k][:, :6] - xi], axis=1)
        u = jnp.maximum(
            jnp.dot(e, w1_ref[...], preferred_element_type=F32) + b1_ref[...], 0.0)
        r1_ref[k] = u
        s_sum = s_sum + jnp.sum(u, axis=0, keepdims=True)
        s_sq = s_sq + jnp.sum(u * u, axis=0, keepdims=True)

    @pl.when(pl.program_id(0) == 0)
    def _():
        st_ref[...] = jnp.zeros_like(st_ref)

    st_ref[...] += jnp.concatenate([s_sum, s_sq], axis=0)


def _k3_call(xj, x16, w1, b1, R):
    return pl.pallas_call(
        _k3_body,
        grid=(T // R,),
        in_specs=[
            pl.BlockSpec((NK, R, 128), lambda i: (0, i, 0)),
            pl.BlockSpec((R, 16), lambda i: (i, 0)),
            pl.BlockSpec((12, 64), lambda i: (0, 0)),
            pl.BlockSpec((1, 64), lambda i: (0, 0)),
        ],
        out_specs=[
            pl.BlockSpec((NK, R, 64), lambda i: (0, i, 0)),
            pl.BlockSpec((2, 64), lambda i: (0, 0)),
        ],
        out_shape=[
            jax.ShapeDtypeStruct((NK, T, 64), F32),
            jax.ShapeDtypeStruct((2, 64), F32),
        ],
    )(xj, x16, w1, b1)


def _k4_body(r1_ref, m_ref, s_ref, g_ref, t_ref, w_ref, b_ref, r2_ref, st_ref):
    y = _bn_apply(r1_ref[...], m_ref[...], s_ref[...], g_ref[...], t_ref[...])
    u = jnp.maximum(jnp.dot(y, w_ref[...], preferred_element_type=F32)
                    + b_ref[...], 0.0)
    r2_ref[...] = u

    @pl.when(pl.program_id(0) == 0)
    def _():
        st_ref[...] = jnp.zeros_like(st_ref)

    st_ref[...] += jnp.concatenate(
        [jnp.sum(u, axis=0, keepdims=True), jnp.sum(u * u, axis=0, keepdims=True)],
        axis=0)


def _k4_call(r1_flat, m, s, gg, tt, w, b, Re):
    small = [pl.BlockSpec((1, 64), lambda i: (0, 0))] * 4
    return pl.pallas_call(
        _k4_body,
        grid=(E // Re,),
        in_specs=[pl.BlockSpec((Re, 64), lambda i: (i, 0))] + small + [
            pl.BlockSpec((64, 64), lambda i: (0, 0)),
            pl.BlockSpec((1, 64), lambda i: (0, 0)),
        ],
        out_specs=[
            pl.BlockSpec((Re, 64), lambda i: (i, 0)),
            pl.BlockSpec((2, 64), lambda i: (0, 0)),
        ],
        out_shape=[
            jax.ShapeDtypeStruct((E, 64), F32),
            jax.ShapeDtypeStruct((2, 64), F32),
        ],
    )(r1_flat, m, s, gg, tt, w, b)


def _k5_body(r2_ref, m_ref, s_ref, g_ref, t_ref, w_ref, b_ref,
             mx_ref, mn_ref, st_ref):
    s_sum = jnp.zeros((1, 64), F32)
    s_sq = jnp.zeros((1, 64), F32)
    mx = None
    for k in range(NK):
        y = _bn_apply(r2_ref[k], m_ref[...], s_ref[...], g_ref[...], t_ref[...])
        u = jnp.maximum(jnp.dot(y, w_ref[...], preferred_element_type=F32)
                        + b_ref[...], 0.0)
        s_sum = s_sum + jnp.sum(u, axis=0, keepdims=True)
        s_sq = s_sq + jnp.sum(u * u, axis=0, keepdims=True)
        if mx is None:
            mx, mn = u, u
        else:
            mx = jnp.maximum(mx, u)
            mn = jnp.minimum(mn, u)
    mx_ref[...] = mx
    mn_ref[...] = mn

    @pl.when(pl.program_id(0) == 0)
    def _():
        st_ref[...] = jnp.zeros_like(st_ref)

    st_ref[...] += jnp.concatenate([s_sum, s_sq], axis=0)


def _k5_call(r2, m, s, gg, tt, w, b, R):
    small = [pl.BlockSpec((1, 64), lambda i: (0, 0))] * 4
    return pl.pallas_call(
        _k5_body,
        grid=(T // R,),
        in_specs=[pl.BlockSpec((NK, R, 64), lambda i: (0, i, 0))] + small + [
            pl.BlockSpec((64, 64), lambda i: (0, 0)),
            pl.BlockSpec((1, 64), lambda i: (0, 0)),
        ],
        out_specs=[
            pl.BlockSpec((R, 64), lambda i: (i, 0)),
            pl.BlockSpec((R, 64), lambda i: (i, 0)),
            pl.BlockSpec((2, 64), lambda i: (0, 0)),
        ],
        out_shape=[
            jax.ShapeDtypeStruct((T, 64), F32),
            jax.ShapeDtypeStruct((T, 64), F32),
            jax.ShapeDtypeStruct((2, 64), F32),
        ],
    )(r2, m, s, gg, tt, w, b)


def _kp_body(x_ref, w11, b11, g11, t11, w12, b12, g12, t12,
             w21, b21, g21, t21, w22, b22, g22, t22, p1_ref, p2_ref):
    x3 = x_ref[:, :3]

    def block(h, w, b, gg, tt):
        u = jnp.dot(h, w[...], preferred_element_type=F32) + b[...]
        m = jnp.mean(u, axis=0, keepdims=True)
        c = u - m
        v = jnp.mean(c * c, axis=0, keepdims=True)
        return jnp.maximum(c / jnp.sqrt(v + EPS) * gg[...] + tt[...], 0.0)

    h = block(x3, w11, b11, g11, t11)
    p1_ref[...] = block(h, w12, b12, g12, t12)
    h = block(x3, w21, b21, g21, t21)
    p2_ref[...] = block(h, w22, b22, g22, t22)


def _kp_call(x16, pos1, pos2):
    args = [x16]
    for (w, b, gg, tt) in list(pos1) + list(pos2):
        args += [w, b.reshape(1, -1), gg.reshape(1, -1), tt.reshape(1, -1)]
    return pl.pallas_call(
        _kp_body,
        out_shape=[
            jax.ShapeDtypeStruct((T, 64), F32),
            jax.ShapeDtypeStruct((T, 256), F32),
        ],
    )(*args)


def _k6_body(mx_ref, mn_ref, pos1_ref, m_ref, s_ref, g_ref, t_ref,
             sw1, sb1, sw2, sb2, h_ref):
    sel = g_ref[...] >= 0.0
    r = jnp.where(sel, mx_ref[...], mn_ref[...])
    conv = _bn_apply(r, m_ref[...], s_ref[...], g_ref[...], t_ref[...])
    h = conv + pos1_ref[...]
    h3 = h.reshape(G, N, 64)
    avg = jnp.mean(h3, axis=1)
    a = jnp.maximum(jnp.dot(avg, sw1[...], preferred_element_type=F32) + sb1[...], 0.0)
    attn = jax.nn.sigmoid(jnp.dot(a, sw2[...], preferred_element_type=F32) + sb2[...])
    h_ref[...] = (h3 * attn[:, None, :]).reshape(T, 64)


def _k6_call(mx, mn, pos1, m, s, gg, tt, se1):
    sw1, sb1, sw2, sb2 = se1
    return pl.pallas_call(
        _k6_body,
        out_shape=jax.ShapeDtypeStruct((T, 64), F32),
    )(mx, mn, pos1, m, s, gg, tt, sw1, sb1.reshape(1, -1), sw2, sb2.reshape(1, -1))


_SC_CH = 256


def _sc_gather(table, idx_flat):
    info = plsc.get_sparse_core_info()
    nw = info.num_cores * info.num_subcores
    bpw = E // nw
    mesh = plsc.VectorSubcoreMesh(core_axis_name="c", subcore_axis_name="s")

    @functools.partial(
        pl.kernel,
        out_type=jax.ShapeDtypeStruct((E, 128), F32),
        mesh=mesh,
        scratch_types=[
            pltpu.VMEM((bpw,), I32),
            pltpu.VMEM((_SC_CH, 128), F32),
            pltpu.SemaphoreType.DMA,
        ],
    )
    def k8(tab_hbm, idx_hbm, out_hbm, idx_v, rows_v, sem):
        wid = lax.axis_index("s") * info.num_cores + lax.axis_index("c")
        base = wid * bpw
        pltpu.sync_copy(idx_hbm.at[pl.ds(base, bpw)], idx_v)

        def chunk(c, carry):
            pltpu.async_copy(
                tab_hbm.at[idx_v.at[pl.ds(c * _SC_CH, _SC_CH)]], rows_v, sem
            ).wait()
            pltpu.sync_copy(rows_v, out_hbm.at[pl.ds(base + c * _SC_CH, _SC_CH)])
            return carry

        lax.fori_loop(0, bpw // _SC_CH, chunk, 0)

    return k8(table, idx_flat)


def _k9_body(hj_ref, h_ref, w_ref, b_ref, mx_ref, mn_ref, st_ref):
    hi = h_ref[:, :64]
    s_sum = jnp.zeros((1, 256), F32)
    s_sq = jnp.zeros((1, 256), F32)
    mx = None
    for k in range(NK):
        e = jnp.concatenate([hi, hj_ref[k][:, :64] - hi], axis=1)
        t = jnp.maximum(jnp.dot(e, w_ref[...], preferred_element_type=F32)
                        + b_ref[...], 0.0)
        s_sum = s_sum + jnp.sum(t, axis=0, keepdims=True)
        s_sq = s_sq + jnp.sum(t * t, axis=0, keepdims=True)
        if mx is None:
            mx, mn = t, t
        else:
            mx = jnp.maximum(mx, t)
            mn = jnp.minimum(mn, t)
    mx_ref[...] = mx
    mn_ref[...] = mn

    @pl.when(pl.program_id(0) == 0)
    def _():
        st_ref[...] = jnp.zeros_like(st_ref)

    st_ref[...] += jnp.concatenate([s_sum, s_sq], axis=0)


def _k9_call(hj, h128, w, b, R):
    return pl.pallas_call(
        _k9_body,
        grid=(T // R,),
        in_specs=[
            pl.BlockSpec((NK, R, 128), lambda i: (0, i, 0)),
            pl.BlockSpec((R, 128), lambda i: (i, 0)),
            pl.BlockSpec((128, 256), lambda i: (0, 0)),
            pl.BlockSpec((1, 256), lambda i: (0, 0)),
        ],
        out_specs=[
            pl.BlockSpec((R, 256), lambda i: (i, 0)),
            pl.BlockSpec((R, 256), lambda i: (i, 0)),
            pl.BlockSpec((2, 256), lambda i: (0, 0)),
        ],
        out_shape=[
            jax.ShapeDtypeStruct((T, 256), F32),
            jax.ShapeDtypeStruct((T, 256), F32),
            jax.ShapeDtypeStruct((2, 256), F32),
        ],
    )(hj, h128, w, b)


def _k10_body(mx_ref, mn_ref, pos2_ref, m_ref, s_ref, g_ref, t_ref,
              sw1, sb1, sw2, sb2, g_out_ref):
    sel = g_ref[...] >= 0.0
    r = jnp.where(sel, mx_ref[...], mn_ref[...])
    h = _bn_apply(r, m_ref[...], s_ref[...], g_ref[...], t_ref[...]) + pos2_ref[...]
    h3 = h.reshape(G, N, 256)
    avg = jnp.mean(h3, axis=1)
    a = jnp.maximum(jnp.dot(avg, sw1[...], preferred_element_type=F32) + sb1[...], 0.0)
    attn = jax.nn.sigmoid(jnp.dot(a, sw2[...], preferred_element_type=F32) + sb2[...])
    g_out_ref[...] = jnp.max(h3 * attn[:, None, :], axis=1)


def _k10_call(mx, mn, pos2, m, s, gg, tt, se2):
    sw1, sb1, sw2, sb2 = se2
    return pl.pallas_call(
        _k10_body,
        out_shape=jax.ShapeDtypeStruct((G, 256), F32),
    )(mx, mn, pos2, m, s, gg, tt, sw1, sb1.reshape(1, -1), sw2, sb2.reshape(1, -1))


def _k11_body(g1_ref, g2_ref, w_ref, b_ref, gg_ref, tt_ref, wo_ref, bo_ref, o_ref):
    d = g2_ref[...] - g1_ref[...]
    u = jnp.maximum(jnp.dot(d, w_ref[...], preferred_element_type=F32) + b_ref[...], 0.0)
    m = jnp.mean(u, axis=0, keepdims=True)
    c = u - m
    v = jnp.mean(c * c, axis=0, keepdims=True)
    hb = c / jnp.sqrt(v + EPS) * gg_ref[...] + tt_ref[...]
    o = jnp.dot(hb, wo_ref[...], preferred_element_type=F32) + bo_ref[...]
    mo = jnp.max(o, axis=1, keepdims=True)
    z = o - mo
    o_ref[...] = z - jnp.log(jnp.sum(jnp.exp(z), axis=1, keepdims=True))


def _k11_call(g1, g2, l1, lout):
    w, b, gg, tt = l1
    wo, bo = lout
    return pl.pallas_call(
        _k11_body,
        out_shape=jax.ShapeDtypeStruct((G, 5), F32),
    )(g1, g2, w, b.reshape(1, -1), gg.reshape(1, -1), tt.reshape(1, -1),
      wo, bo.reshape(1, -1))


def _bn_ms(st, count):
    m = st[0:1] / count
    v = st[1:2] / count - m * m
    return m, jnp.sqrt(v + EPS)


def _branch_fast(xf, p):
    x16 = jnp.pad(xf, ((0, 0), (0, 10)))
    x3d = x16.reshape(G, N, 16)
    xT3d = jnp.transpose(x3d, (0, 2, 1))

    pos1, pos2 = _kp_call(x16, p['pos1'], p['pos2'])

    (w1, b1, g1, t1), (w2, b2, g2, t2), (w3, b3, g3, t3) = p['conv1']
    idx1 = _ksel_call(x3d, xT3d, 256)
    x128 = jnp.pad(x16, ((0, 0), (0, 112)))
    xj = _sc_gather(x128, idx1.reshape(E))
    r1, st1 = _k3_call(xj.reshape(NK, T, 128), x16, w1, b1.reshape(1, -1), 256)

    m1, s1 = _bn_ms(st1, float(E))
    r2_flat, st2 = _k4_call(r1.reshape(E, 64), m1, s1, g1.reshape(1, -1),
                            t1.reshape(1, -1), w2, b2.reshape(1, -1), 2048)

    m2, s2 = _bn_ms(st2, float(E))
    mx3, mn3, st3 = _k5_call(r2_flat.reshape(NK, T, 64), m2, s2, g2.reshape(1, -1),
                             t2.reshape(1, -1), w3, b3.reshape(1, -1), 512)

    m3, s3 = _bn_ms(st3, float(E))
    h = _k6_call(mx3, mn3, pos1, m3, s3, g3.reshape(1, -1), t3.reshape(1, -1),
                 p['se1'])

    h3d = h.reshape(G, N, 64)
    hT3d = jnp.transpose(h3d, (0, 2, 1))
    idx2 = _ksel_call(h3d, hT3d, 256)

    h128 = jnp.pad(h, ((0, 0), (0, 64)))
    hj = _sc_gather(h128, idx2.reshape(E))

    (w2c, b2c, g2c, t2c), = p['conv2']
    mx2, mn2, st4 = _k9_call(hj.reshape(NK, T, 128), h128, w2c,
                             b2c.reshape(1, -1), 256)

    m4, s4 = _bn_ms(st4, float(E))
    return _k10_call(mx2, mn2, pos2, m4, s4, g2c.reshape(1, -1),
                     t2c.reshape(1, -1), p['se2'])


def kernel(x, x2, batch, batch2, y, params):
    del batch, batch2, y
    gA = _branch_fast(x, params)
    gB = _branch_fast(x2, params)
    return _k11_call(gA, gB, params['mlp2_l1'], params['mlp2_out'])
